# Initial kernel scaffold; baseline (speedup 1.0000x reference)
import jax
import jax.numpy as jnp
from jax.experimental import pallas as pl


def kernel(cls_logits, box_preds, centerness, gt_boxes, gt_labels):
    raise NotImplementedError("write your pallas kernel here")


# speedup vs baseline: 7.1822x; 7.1822x over previous
import functools

import jax
import jax.numpy as jnp
from jax.experimental import pallas as pl
from jax.experimental.pallas import tpu as pltpu

B, N, C, G = 8, 4096, 80, 16
IMG = 512.0
H = 64


def _loss_body(cls_ref, box_ref, ctr_ref, gtb_ref, gtl_ref, out_ref):
    b = pl.program_id(0)

    logits = cls_ref[0]
    box_preds = box_ref[0]
    z = ctr_ref[0]
    gtb = gtb_ref[0]
    gtl = gtl_ref[0, 0]

    lin = jax.lax.broadcasted_iota(jnp.int32, (N, 1), 0)
    cx = ((lin & (H - 1)).astype(jnp.float32) + 0.5) / H
    cy = ((lin >> 6).astype(jnp.float32) + 0.5) / H

    nb = gtb / IMG
    gt_cx = (nb[:, 0] + nb[:, 2]) * 0.5
    gt_cy = (nb[:, 1] + nb[:, 3]) * 0.5

    dx = cx - gt_cx[None, :]
    dy = cy - gt_cy[None, :]
    dist = jnp.sqrt(dx * dx + dy * dy + 1e-12)
    min_dist = jnp.min(dist, axis=1, keepdims=True)
    iota_g = jax.lax.broadcasted_iota(jnp.int32, (N, G), 1)
    best = jnp.min(jnp.where(dist == min_dist, iota_g, G), axis=1,
                   keepdims=True)
    pos = min_dist < (1.5 / H)
    onehot = best == iota_g

    labels_best = jnp.sum(jnp.where(onehot, gtl[None, :], 0), axis=1,
                          keepdims=True)
    cls_target = jnp.where(pos, labels_best, 0)
    iota_c = jax.lax.broadcasted_iota(jnp.int32, (N, C), 1)
    x_t = jnp.sum(jnp.where(iota_c == cls_target, logits, 0.0), axis=1,
                  keepdims=True)
    mx = jnp.max(logits, axis=1, keepdims=True)
    lse = mx + jnp.log(jnp.sum(jnp.exp(logits - mx), axis=1, keepdims=True))
    ce = lse - x_t
    pt = jnp.exp(-ce)
    om = 1.0 - pt
    fl = 0.25 * om * om * ce
    cls_mean = jnp.sum(fl) * (1.0 / N)

    gt_w = nb[:, 2] - nb[:, 0]
    gt_h = nb[:, 3] - nb[:, 1]

    def gather(col):
        return jnp.sum(jnp.where(onehot, col[None, :], 0.0), axis=1,
                       keepdims=True)

    g_cx = gather(gt_cx)
    g_cy = gather(gt_cy)
    g_w = gather(gt_w)
    g_h = gather(gt_h)

    m = pos.astype(jnp.float32)
    cnt = jnp.sum(m)

    gb = jnp.concatenate([g_cx, g_cy, g_w, g_h], axis=1)
    ad = jnp.abs(box_preds - gb)
    sl1 = jnp.where(ad < 1.0, 0.5 * ad * ad, ad - 0.5)
    box_sum = jnp.sum(sl1 * m)
    box_per = box_sum / jnp.maximum(cnt * 4.0, 1.0)
    box_term = jnp.where(cnt > 0, box_per, 0.0)

    bx0 = gather(nb[:, 0])
    bx1 = gather(nb[:, 1])
    bx2 = gather(nb[:, 2])
    bx3 = gather(nb[:, 3])
    l = jnp.maximum(cx - bx0, 1e-06)
    r = jnp.maximum(bx2 - cx, 1e-06)
    t = jnp.maximum(cy - bx1, 1e-06)
    bt = jnp.maximum(bx3 - cy, 1e-06)
    ratio = (jnp.minimum(l, r) / jnp.maximum(l, r)) * (jnp.minimum(t, bt) /
                                                       jnp.maximum(t, bt))
    ctr_t = jnp.clip(jnp.sqrt(ratio), 0.0, 1.0)
    bce = jnp.maximum(z, 0.0) - z * ctr_t + jnp.log1p(jnp.exp(-jnp.abs(z)))
    ctr_sum = jnp.sum(bce * m)
    ctr_term = jnp.where(cnt > 0, ctr_sum / jnp.maximum(cnt, 1.0), 0.0)

    part = jnp.concatenate(
        [jnp.reshape(cls_mean, (1, 1)), jnp.reshape(box_term, (1, 1)),
         jnp.reshape(ctr_term, (1, 1))], axis=1)

    @pl.when(b == 0)
    def _():
        out_ref[...] = jnp.zeros_like(out_ref)

    out_ref[...] += part


@jax.jit
def kernel(cls_logits, box_preds, centerness, gt_boxes, gt_labels):
    gtl3 = gt_labels.reshape(B, 1, G)
    totals = pl.pallas_call(
        _loss_body,
        grid=(B,),
        in_specs=[
            pl.BlockSpec((1, N, C), lambda b: (b, 0, 0)),
            pl.BlockSpec((1, N, 4), lambda b: (b, 0, 0)),
            pl.BlockSpec((1, N, 1), lambda b: (b, 0, 0)),
            pl.BlockSpec((1, G, 4), lambda b: (b, 0, 0)),
            pl.BlockSpec((1, 1, G), lambda b: (b, 0, 0)),
        ],
        out_specs=pl.BlockSpec((1, 3), lambda b: (0, 0)),
        out_shape=jax.ShapeDtypeStruct((1, 3), jnp.float32),
    )(cls_logits, box_preds, centerness, gt_boxes, gtl3)
    tc = totals[0, 0]
    tb = totals[0, 1]
    tr = totals[0, 2]
    loss = tc / B + 5.0 * tb / B + 1.0 * tr / B
    return (loss, tc / B, tb / B, tr / B)


# speedup vs baseline: 24.2986x; 3.3832x over previous
import jax
import jax.numpy as jnp
from jax.experimental import pallas as pl
from jax.experimental.pallas import tpu as pltpu

B, N, C, G = 8, 4096, 80, 16
IMG = 512.0
H = 64
SUB, LANE = 32, 128


def _loss_body(gtb_ref, gtl_ref, cls_ref, box_ref, ctr_ref, out_ref):
    b = pl.program_id(0)
    shp = (SUB, LANE)

    row = jax.lax.broadcasted_iota(jnp.int32, shp, 0)
    col = jax.lax.broadcasted_iota(jnp.int32, shp, 1)
    lin = row * LANE + col
    cx = ((lin & (H - 1)).astype(jnp.float32) + 0.5) * (1.0 / H)
    cy = (lin >> 6).astype(jnp.float32) * (1.0 / H) + (0.5 / H)

    minv = jnp.full(shp, jnp.inf, dtype=jnp.float32)
    lab = jnp.zeros(shp, dtype=jnp.int32)
    b0 = jnp.zeros(shp, dtype=jnp.float32)
    b1 = jnp.zeros(shp, dtype=jnp.float32)
    b2 = jnp.zeros(shp, dtype=jnp.float32)
    b3 = jnp.zeros(shp, dtype=jnp.float32)
    for g in range(G):
        x0 = gtb_ref[0, g, 0] * (1.0 / IMG)
        y0 = gtb_ref[0, g, 1] * (1.0 / IMG)
        x1 = gtb_ref[0, g, 2] * (1.0 / IMG)
        y1 = gtb_ref[0, g, 3] * (1.0 / IMG)
        dx = cx - (x0 + x1) * 0.5
        dy = cy - (y0 + y1) * 0.5
        dist = jnp.sqrt(dx * dx + dy * dy + 1e-12)
        upd = dist < minv
        minv = jnp.where(upd, dist, minv)
        lab = jnp.where(upd, gtl_ref[0, 0, g], lab)
        b0 = jnp.where(upd, x0, b0)
        b1 = jnp.where(upd, y0, b1)
        b2 = jnp.where(upd, x1, b2)
        b3 = jnp.where(upd, y1, b3)

    pos = minv < (1.5 / H)
    m = pos.astype(jnp.float32)
    cnt = jnp.sum(m)

    tgt = jnp.where(pos, lab, 0)
    se = jnp.zeros(shp, dtype=jnp.float32)
    xt = jnp.zeros(shp, dtype=jnp.float32)
    for c in range(C):
        lc = cls_ref[0, c]
        se = se + jnp.exp(lc)
        xt = jnp.where(tgt == c, lc, xt)
    ce = jnp.log(se) - xt
    pt = jnp.exp(-ce)
    om = 1.0 - pt
    fl = 0.25 * om * om * ce
    cls_mean = jnp.sum(fl) * (1.0 / N)

    g_cx = (b0 + b2) * 0.5
    g_cy = (b1 + b3) * 0.5
    g_w = b2 - b0
    g_h = b3 - b1
    sl1 = jnp.zeros(shp, dtype=jnp.float32)
    for i, gc in enumerate((g_cx, g_cy, g_w, g_h)):
        ad = jnp.abs(box_ref[0, i] - gc)
        sl1 = sl1 + jnp.where(ad < 1.0, 0.5 * ad * ad, ad - 0.5)
    box_sum = jnp.sum(sl1 * m)
    box_term = jnp.where(cnt > 0, box_sum / jnp.maximum(cnt * 4.0, 1.0), 0.0)

    l = jnp.maximum(cx - b0, 1e-06)
    r = jnp.maximum(b2 - cx, 1e-06)
    t = jnp.maximum(cy - b1, 1e-06)
    bt = jnp.maximum(b3 - cy, 1e-06)
    ratio = (jnp.minimum(l, r) / jnp.maximum(l, r)) * (jnp.minimum(t, bt) /
                                                       jnp.maximum(t, bt))
    ctr_t = jnp.clip(jnp.sqrt(ratio), 0.0, 1.0)
    z = ctr_ref[0]
    bce = jnp.maximum(z, 0.0) - z * ctr_t + jnp.log1p(jnp.exp(-jnp.abs(z)))
    ctr_sum = jnp.sum(bce * m)
    ctr_term = jnp.where(cnt > 0, ctr_sum / jnp.maximum(cnt, 1.0), 0.0)

    part = jnp.concatenate(
        [jnp.reshape(cls_mean, (1, 1)), jnp.reshape(box_term, (1, 1)),
         jnp.reshape(ctr_term, (1, 1))], axis=1)

    @pl.when(b == 0)
    def _():
        out_ref[...] = jnp.zeros_like(out_ref)

    out_ref[...] += part


@jax.jit
def kernel(cls_logits, box_preds, centerness, gt_boxes, gt_labels):
    clsT = cls_logits.transpose(0, 2, 1).reshape(B, C, SUB, LANE)
    boxT = box_preds.transpose(0, 2, 1).reshape(B, 4, SUB, LANE)
    ctrT = centerness.reshape(B, SUB, LANE)
    gtl3 = gt_labels.reshape(B, 1, G)
    totals = pl.pallas_call(
        _loss_body,
        grid=(B,),
        in_specs=[
            pl.BlockSpec((1, G, 4), lambda b: (b, 0, 0),
                         memory_space=pltpu.SMEM),
            pl.BlockSpec((1, 1, G), lambda b: (b, 0, 0),
                         memory_space=pltpu.SMEM),
            pl.BlockSpec((1, C, SUB, LANE), lambda b: (b, 0, 0, 0)),
            pl.BlockSpec((1, 4, SUB, LANE), lambda b: (b, 0, 0, 0)),
            pl.BlockSpec((1, SUB, LANE), lambda b: (b, 0, 0)),
        ],
        out_specs=pl.BlockSpec((1, 3), lambda b: (0, 0)),
        out_shape=jax.ShapeDtypeStruct((1, 3), jnp.float32),
    )(gt_boxes, gtl3, clsT, boxT, ctrT)
    tc = totals[0, 0]
    tb = totals[0, 1]
    tr = totals[0, 2]
    loss = tc / B + 5.0 * tb / B + 1.0 * tr / B
    return (loss, tc / B, tb / B, tr / B)
